# baseline (speedup 1.0000x reference)
import functools

import jax
import jax.numpy as jnp
from jax import lax
from jax.experimental import pallas as pl
from jax.experimental.pallas import tpu as pltpu
from jax.experimental.pallas import tpu_sc as plsc

N = 2048
D = 2048
E = 8
F = 8192
T = 128
NB = (N // T) + (E - 1)
NBT = NB * T
FT = 512
K = F // FT
MW = 32
AUX_W = 0.01


def _route_body(x_ref, gw_ref, dest_ref, tp_ref, meta_ref, aux_ref):
    xf = x_ref[...]
    gw = gw_ref[...]
    logits = lax.dot_general(xf, gw, (((1,), (1,)), ((), ())),
                             preferred_element_type=jnp.float32)
    m = jnp.max(logits, axis=1, keepdims=True)
    p = jnp.exp(logits - m)
    probs = p / jnp.sum(p, axis=1, keepdims=True)
    top_p = jnp.max(probs, axis=1, keepdims=True)
    lane = lax.broadcasted_iota(jnp.int32, (N, E), 1)
    idx = jnp.min(jnp.where(probs == top_p, lane, E), axis=1, keepdims=True)
    onehot = (lane == idx).astype(jnp.float32)

    counts = jnp.sum(onehot, axis=0, keepdims=True)
    imp = jnp.sum(probs, axis=0, keepdims=True)
    aux_ref[...] = (jnp.sum((counts / N) * (imp / N), keepdims=True)
                    * E * AUX_W).reshape(1, 1)

    r_i = lax.broadcasted_iota(jnp.int32, (N, N), 0)
    c_i = lax.broadcasted_iota(jnp.int32, (N, N), 1)
    lt = (c_i < r_i).astype(jnp.float32)
    rank_te = lax.dot_general(lt, onehot, (((1,), (0,)), ((), ())),
                              preferred_element_type=jnp.float32)
    rank = jnp.sum(rank_te * onehot, axis=1, keepdims=True)

    counts_i = counts.astype(jnp.int32)
    nblocks = (counts_i + (T - 1)) // T
    e_r = lax.broadcasted_iota(jnp.int32, (E, E), 0)
    e_c = lax.broadcasted_iota(jnp.int32, (E, E), 1)
    lt8 = (e_r < e_c).astype(jnp.float32)
    blkcum = lax.dot_general(nblocks.astype(jnp.float32), lt8,
                             (((1,), (0,)), ((), ())),
                             preferred_element_type=jnp.float32)
    blkcum = blkcum.astype(jnp.int32)
    base = (blkcum * T).astype(jnp.float32)
    dest = rank + jnp.sum(onehot * base, axis=1, keepdims=True)
    dest_ref[...] = dest.astype(jnp.int32)
    tp_ref[...] = jnp.broadcast_to(top_p, (N, 128))

    nvalid = jnp.sum(nblocks)
    blkend = blkcum + nblocks
    s_l = lax.broadcasted_iota(jnp.int32, (1, MW), 1)
    s_eff_l = jnp.minimum(s_l, nvalid - 1)
    cnt = jnp.zeros((1, MW), jnp.int32)
    for e in range(E):
        cnt = cnt + (blkend[0, e] <= s_eff_l).astype(jnp.int32)
    meta = jnp.where(s_l < MW - 1, cnt, nvalid)
    meta_ref[...] = meta


def _route(x_flat, gate_w):
    return pl.pallas_call(
        _route_body,
        out_shape=(
            jax.ShapeDtypeStruct((N, 1), jnp.int32),
            jax.ShapeDtypeStruct((N, 128), jnp.float32),
            jax.ShapeDtypeStruct((1, MW), jnp.int32),
            jax.ShapeDtypeStruct((1, 1), jnp.float32),
        ),
    )(x_flat, gate_w)


def _sc_scatter(x_flat, tp_rep, dest):
    info = plsc.get_sparse_core_info()
    nc, ns = info.num_cores, info.num_subcores
    nw = nc * ns
    per_w = N // nw
    chunk = 32
    nch = per_w // chunk

    mesh = plsc.VectorSubcoreMesh(core_axis_name="c", subcore_axis_name="s")

    @functools.partial(
        pl.kernel, mesh=mesh,
        out_type=(
            jax.ShapeDtypeStruct((NBT, D), jnp.float32),
            jax.ShapeDtypeStruct((NBT, 128), jnp.float32),
        ),
        scratch_types=[
            pltpu.VMEM((chunk, D), jnp.float32),
            pltpu.VMEM((chunk, 128), jnp.float32),
            pltpu.VMEM((chunk,), jnp.int32),
            pltpu.SemaphoreType.DMA,
            pltpu.SemaphoreType.DMA,
        ],
    )
    def kb(x_hbm, tp_hbm, dest_hbm, xs_hbm, tps_hbm,
           xbuf, tbuf, ibuf, sem1, sem2):
        wid = lax.axis_index("s") * nc + lax.axis_index("c")
        for j in range(nch):
            base = wid * per_w + j * chunk
            pltpu.sync_copy(dest_hbm.at[pl.ds(base, chunk)], ibuf)
            pltpu.sync_copy(x_hbm.at[pl.ds(base, chunk)], xbuf)
            pltpu.sync_copy(tp_hbm.at[pl.ds(base, chunk)], tbuf)
            pltpu.async_copy(xbuf, xs_hbm.at[ibuf], sem1).wait()
            pltpu.async_copy(tbuf, tps_hbm.at[ibuf], sem2).wait()

    return kb(x_flat, tp_rep, dest)


def _ffn_body(meta_ref, w1_ref, w3_ref, w2_ref, tp_ref, xs_ref,
              out_any, acc_v, sem_out):
    k = pl.program_id(0)
    b = pl.program_id(1)
    nvalid = meta_ref[MW - 1]

    @pl.when(b < nvalid)
    def _compute():
        rows = pl.ds(b * T, T)
        xb = xs_ref[...].astype(jnp.bfloat16)
        w1t = w1_ref[0].astype(jnp.bfloat16)
        w3t = w3_ref[0].astype(jnp.bfloat16)
        w2t = w2_ref[0].astype(jnp.bfloat16)
        a1 = lax.dot_general(xb, w1t, (((1,), (1,)), ((), ())),
                             preferred_element_type=jnp.float32)
        a3 = lax.dot_general(xb, w3t, (((1,), (1,)), ((), ())),
                             preferred_element_type=jnp.float32)
        h = a1 * jax.nn.sigmoid(a1) * a3
        contrib = lax.dot_general(h.astype(jnp.bfloat16), w2t,
                                  (((1,), (1,)), ((), ())),
                                  preferred_element_type=jnp.float32)

        @pl.when(k == 0)
        def _init():
            acc_v[rows, :] = contrib

        @pl.when(k != 0)
        def _acc():
            acc_v[rows, :] = acc_v[rows, :] + contrib

        @pl.when(k == K - 1)
        def _emit():
            tp = tp_ref[rows, 0:1]
            acc_v[rows, :] = acc_v[rows, :] * tp
            pltpu.make_async_copy(acc_v.at[rows], out_any.at[rows],
                                  sem_out).start()
            pltpu.make_async_copy(acc_v.at[rows], out_any.at[rows],
                                  sem_out).wait()


def _ffn(xs, tps, w1, w2, w3, meta):
    grid_spec = pltpu.PrefetchScalarGridSpec(
        num_scalar_prefetch=1,
        grid=(K, NB),
        in_specs=[
            pl.BlockSpec((1, FT, D), lambda k, b, meta: (meta[b], k, 0)),
            pl.BlockSpec((1, FT, D), lambda k, b, meta: (meta[b], k, 0)),
            pl.BlockSpec((1, D, FT), lambda k, b, meta: (meta[b], 0, k)),
            pl.BlockSpec((NBT, 128), lambda k, b, meta: (0, 0)),
            pl.BlockSpec((T, D),
                         lambda k, b, meta: (jnp.minimum(b, meta[MW - 1] - 1),
                                             0)),
        ],
        out_specs=pl.BlockSpec(memory_space=pl.ANY),
        scratch_shapes=[
            pltpu.VMEM((NBT, D), jnp.float32),
            pltpu.SemaphoreType.DMA,
        ],
    )
    return pl.pallas_call(
        _ffn_body,
        grid_spec=grid_spec,
        out_shape=jax.ShapeDtypeStruct((NBT, D), jnp.float32),
        compiler_params=pltpu.CompilerParams(
            dimension_semantics=("arbitrary", "arbitrary")),
    )(meta, w1, w3, w2, tps, xs)


def _sc_gather(out_c, dest):
    info = plsc.get_sparse_core_info()
    nc, ns = info.num_cores, info.num_subcores
    nw = nc * ns
    per_w = N // nw
    chunk = 32
    nch = per_w // chunk

    mesh = plsc.VectorSubcoreMesh(core_axis_name="c", subcore_axis_name="s")

    @functools.partial(
        pl.kernel, mesh=mesh,
        out_type=jax.ShapeDtypeStruct((N, D), jnp.float32),
        scratch_types=[
            pltpu.VMEM((chunk, D), jnp.float32),
            pltpu.VMEM((chunk,), jnp.int32),
            pltpu.SemaphoreType.DMA,
        ],
    )
    def kd(outc_hbm, dest_hbm, outf_hbm, xbuf, ibuf, sem):
        wid = lax.axis_index("s") * nc + lax.axis_index("c")
        for j in range(nch):
            base = wid * per_w + j * chunk
            pltpu.sync_copy(dest_hbm.at[pl.ds(base, chunk)], ibuf)
            pltpu.async_copy(outc_hbm.at[ibuf], xbuf, sem).wait()
            pltpu.sync_copy(xbuf, outf_hbm.at[pl.ds(base, chunk)])

    return kd(out_c, dest)


@jax.jit
def kernel(x, gate_w, w1, w2, w3):
    x_flat = x.reshape(N, D)
    dest2d, tp_rep, meta, aux = _route(x_flat, gate_w)
    dest = dest2d.reshape(N)
    xs, tps = _sc_scatter(x_flat, tp_rep, dest)
    out_c = _ffn(xs, tps, w1, w2, w3, meta.reshape(MW))
    out = _sc_gather(out_c, dest)
    return out.reshape(x.shape), aux.reshape(())


# speedup vs baseline: 1.1979x; 1.1979x over previous
import functools

import jax
import jax.numpy as jnp
from jax import lax
from jax.experimental import pallas as pl
from jax.experimental.pallas import tpu as pltpu
from jax.experimental.pallas import tpu_sc as plsc

N = 2048
D = 2048
E = 8
F = 8192
T = 128
NB = (N // T) + (E - 1)
NBT = NB * T
FT = 256
K = F // FT
MW = 32
AUX_W = 0.01


def _route_body(x_ref, gw_ref, dest_ref, tp_ref, meta_ref, aux_ref):
    xf = x_ref[...]
    gw = gw_ref[...]
    logits = lax.dot_general(xf, gw, (((1,), (1,)), ((), ())),
                             preferred_element_type=jnp.float32)
    m = jnp.max(logits, axis=1, keepdims=True)
    p = jnp.exp(logits - m)
    probs = p / jnp.sum(p, axis=1, keepdims=True)
    top_p = jnp.max(probs, axis=1, keepdims=True)
    lane = lax.broadcasted_iota(jnp.int32, (N, E), 1)
    idx = jnp.min(jnp.where(probs == top_p, lane, E), axis=1, keepdims=True)
    onehot = (lane == idx).astype(jnp.float32)

    counts = jnp.sum(onehot, axis=0, keepdims=True)
    imp = jnp.sum(probs, axis=0, keepdims=True)
    aux_ref[...] = (jnp.sum((counts / N) * (imp / N), keepdims=True)
                    * E * AUX_W).reshape(1, 1)

    r_i = lax.broadcasted_iota(jnp.int32, (N, N), 0)
    c_i = lax.broadcasted_iota(jnp.int32, (N, N), 1)
    lt = (c_i < r_i).astype(jnp.float32)
    rank_te = lax.dot_general(lt, onehot, (((1,), (0,)), ((), ())),
                              preferred_element_type=jnp.float32)
    rank = jnp.sum(rank_te * onehot, axis=1, keepdims=True)

    counts_i = counts.astype(jnp.int32)
    nblocks = (counts_i + (T - 1)) // T
    e_r = lax.broadcasted_iota(jnp.int32, (E, E), 0)
    e_c = lax.broadcasted_iota(jnp.int32, (E, E), 1)
    lt8 = (e_r < e_c).astype(jnp.float32)
    blkcum = lax.dot_general(nblocks.astype(jnp.float32), lt8,
                             (((1,), (0,)), ((), ())),
                             preferred_element_type=jnp.float32)
    blkcum = blkcum.astype(jnp.int32)
    base = (blkcum * T).astype(jnp.float32)
    dest = rank + jnp.sum(onehot * base, axis=1, keepdims=True)
    dest_ref[...] = dest.astype(jnp.int32)
    tp_ref[...] = jnp.broadcast_to(top_p, (N, 128))

    zpad = jnp.zeros((1, MW - 2 * E), jnp.int32)
    meta_ref[...] = jnp.concatenate([blkcum, nblocks, zpad], axis=1)


def _route(x_flat, gate_w):
    return pl.pallas_call(
        _route_body,
        out_shape=(
            jax.ShapeDtypeStruct((N, 1), jnp.int32),
            jax.ShapeDtypeStruct((N, 128), jnp.float32),
            jax.ShapeDtypeStruct((1, MW), jnp.int32),
            jax.ShapeDtypeStruct((1, 1), jnp.float32),
        ),
    )(x_flat, gate_w)


def _sc_scatter(x_flat, tp_rep, dest):
    info = plsc.get_sparse_core_info()
    nc, ns = info.num_cores, info.num_subcores
    nw = nc * ns
    per_w = N // nw
    chunk = 32
    nch = per_w // chunk

    mesh = plsc.VectorSubcoreMesh(core_axis_name="c", subcore_axis_name="s")

    @functools.partial(
        pl.kernel, mesh=mesh,
        out_type=(
            jax.ShapeDtypeStruct((NBT, D), jnp.float32),
            jax.ShapeDtypeStruct((NBT, 128), jnp.float32),
        ),
        scratch_types=[
            pltpu.VMEM((chunk, D), jnp.float32),
            pltpu.VMEM((chunk, 128), jnp.float32),
            pltpu.VMEM((chunk,), jnp.int32),
            pltpu.SemaphoreType.DMA,
            pltpu.SemaphoreType.DMA,
        ],
    )
    def kb(x_hbm, tp_hbm, dest_hbm, xs_hbm, tps_hbm,
           xbuf, tbuf, ibuf, sem1, sem2):
        wid = lax.axis_index("s") * nc + lax.axis_index("c")
        for j in range(nch):
            base = wid * per_w + j * chunk
            pltpu.sync_copy(dest_hbm.at[pl.ds(base, chunk)], ibuf)
            pltpu.sync_copy(x_hbm.at[pl.ds(base, chunk)], xbuf)
            pltpu.sync_copy(tp_hbm.at[pl.ds(base, chunk)], tbuf)
            pltpu.async_copy(xbuf, xs_hbm.at[ibuf], sem1).wait()
            pltpu.async_copy(tbuf, tps_hbm.at[ibuf], sem2).wait()

    return kb(x_flat, tp_rep, dest)


def _ffn_body(meta_ref, w1_ref, w3_ref, w2_ref, tp_ref, xs_any,
              out_any, xs_v, acc_v, sem_in, sem_out):
    e = pl.program_id(0)
    k = pl.program_id(1)
    base = meta_ref[e]
    nb = meta_ref[E + e]

    @pl.when((e == 0) & (k == 0))
    def _load_x():
        pltpu.make_async_copy(xs_any, xs_v, sem_in).start()
        pltpu.make_async_copy(xs_any, xs_v, sem_in).wait()

    w1t = w1_ref[0]
    w3t = w3_ref[0]
    w2t = w2_ref[0]

    def blk(j, carry):
        rows_x = pl.ds((base + j) * T, T)
        rows_a = pl.ds(j * T, T)
        xb = xs_v[rows_x, :]
        a1 = lax.dot_general(xb, w1t, (((1,), (1,)), ((), ())),
                             preferred_element_type=jnp.float32)
        a3 = lax.dot_general(xb, w3t, (((1,), (1,)), ((), ())),
                             preferred_element_type=jnp.float32)
        h = a1 * jax.nn.sigmoid(a1) * a3
        contrib = lax.dot_general(h, w2t, (((1,), (1,)), ((), ())),
                                  preferred_element_type=jnp.float32)
        acc_v[rows_a, :] = jnp.where(k == 0, contrib,
                                     acc_v[rows_a, :] + contrib)
        return carry

    lax.fori_loop(0, nb, blk, 0)

    @pl.when(k == K - 1)
    def _emit():
        def emit_blk(j, carry):
            rows_x = pl.ds((base + j) * T, T)
            rows_a = pl.ds(j * T, T)
            tp = tp_ref[rows_x, 0:1]
            acc_v[rows_a, :] = acc_v[rows_a, :] * tp
            pltpu.make_async_copy(acc_v.at[rows_a], out_any.at[rows_x],
                                  sem_out).start()
            pltpu.make_async_copy(acc_v.at[rows_a], out_any.at[rows_x],
                                  sem_out).wait()
            return carry
        lax.fori_loop(0, nb, emit_blk, 0)


def _ffn(xs, tps, w1, w2, w3, meta):
    grid_spec = pltpu.PrefetchScalarGridSpec(
        num_scalar_prefetch=1,
        grid=(E, K),
        in_specs=[
            pl.BlockSpec((1, FT, D), lambda e, k, meta: (e, k, 0)),
            pl.BlockSpec((1, FT, D), lambda e, k, meta: (e, k, 0)),
            pl.BlockSpec((1, D, FT), lambda e, k, meta: (e, 0, k)),
            pl.BlockSpec((NBT, 128), lambda e, k, meta: (0, 0)),
            pl.BlockSpec(memory_space=pl.ANY),
        ],
        out_specs=pl.BlockSpec(memory_space=pl.ANY),
        scratch_shapes=[
            pltpu.VMEM((NBT, D), jnp.float32),
            pltpu.VMEM((N, D), jnp.float32),
            pltpu.SemaphoreType.DMA,
            pltpu.SemaphoreType.DMA,
        ],
    )
    return pl.pallas_call(
        _ffn_body,
        grid_spec=grid_spec,
        out_shape=jax.ShapeDtypeStruct((NBT, D), jnp.float32),
        compiler_params=pltpu.CompilerParams(
            dimension_semantics=("arbitrary", "arbitrary")),
    )(meta, w1, w3, w2, tps, xs)


def _sc_gather(out_c, dest):
    info = plsc.get_sparse_core_info()
    nc, ns = info.num_cores, info.num_subcores
    nw = nc * ns
    per_w = N // nw
    chunk = 32
    nch = per_w // chunk

    mesh = plsc.VectorSubcoreMesh(core_axis_name="c", subcore_axis_name="s")

    @functools.partial(
        pl.kernel, mesh=mesh,
        out_type=jax.ShapeDtypeStruct((N, D), jnp.float32),
        scratch_types=[
            pltpu.VMEM((chunk, D), jnp.float32),
            pltpu.VMEM((chunk,), jnp.int32),
            pltpu.SemaphoreType.DMA,
        ],
    )
    def kd(outc_hbm, dest_hbm, outf_hbm, xbuf, ibuf, sem):
        wid = lax.axis_index("s") * nc + lax.axis_index("c")
        for j in range(nch):
            base = wid * per_w + j * chunk
            pltpu.sync_copy(dest_hbm.at[pl.ds(base, chunk)], ibuf)
            pltpu.async_copy(outc_hbm.at[ibuf], xbuf, sem).wait()
            pltpu.sync_copy(xbuf, outf_hbm.at[pl.ds(base, chunk)])

    return kd(out_c, dest)


@jax.jit
def kernel(x, gate_w, w1, w2, w3):
    x_flat = x.reshape(N, D)
    dest2d, tp_rep, meta, aux = _route(x_flat, gate_w)
    dest = dest2d.reshape(N)
    xs, tps = _sc_scatter(x_flat, tp_rep, dest)
    out_c = _ffn(xs, tps, w1, w2, w3, meta.reshape(MW))
    out = _sc_gather(out_c, dest)
    return out.reshape(x.shape), aux.reshape(())
